# Initial kernel scaffold; baseline (speedup 1.0000x reference)
import jax
import jax.numpy as jnp
from jax.experimental import pallas as pl


def kernel(x, pos, angle, keylen, hook, morse, params, edge_index, z, batch, nuc_index):
    raise NotImplementedError("write your pallas kernel here")


# speedup vs baseline: 5.6810x; 5.6810x over previous
import functools

import jax
import jax.numpy as jnp
import numpy as np
from jax import lax
from jax.experimental import pallas as pl
from jax.experimental.pallas import tpu as pltpu
from jax.experimental.pallas import tpu_sc as plsc

N_NODES = 10008
N_EDGES = 320256
B = 1112
NF = 64

EP = 327680
E_PER_TILE = EP // 32
NPAD = 10240
BPAD = 1536

_LN2 = float(np.log(2.0))


def _sp(x):
    return jnp.maximum(x, 0.0) + jnp.log1p(jnp.exp(-jnp.abs(x))) - _LN2


def _mesh():
    return plsc.VectorSubcoreMesh(core_axis_name="c", subcore_axis_name="s")


def _edge_meta_body(pos_hbm, z_hbm, src_hbm, dst_hbm, d2_hbm, pair_hbm,
                    pos_v, z_v, src_v, dst_v, d2_v, pair_v):
    wid = lax.axis_index("s") * 2 + lax.axis_index("c")
    base = pl.multiple_of(wid * E_PER_TILE, E_PER_TILE)
    pltpu.sync_copy(pos_hbm, pos_v)
    pltpu.sync_copy(z_hbm, z_v)
    pltpu.sync_copy(src_hbm.at[pl.ds(base, E_PER_TILE)], src_v)
    pltpu.sync_copy(dst_hbm.at[pl.ds(base, E_PER_TILE)], dst_v)

    def body(g, _):
        off = g * 16
        sv = src_v[pl.ds(off, 16)]
        dv = dst_v[pl.ds(off, 16)]
        s4 = sv * 4
        d4 = dv * 4
        xs = plsc.load_gather(pos_v, [s4])
        ys = plsc.load_gather(pos_v, [s4 + 1])
        zs = plsc.load_gather(pos_v, [s4 + 2])
        xd = plsc.load_gather(pos_v, [d4])
        yd = plsc.load_gather(pos_v, [d4 + 1])
        zd = plsc.load_gather(pos_v, [d4 + 2])
        dx = xs - xd
        dy = ys - yd
        dz = zs - zd
        d2_v[pl.ds(off, 16)] = dx * dx + dy * dy + dz * dz
        zsv = plsc.load_gather(z_v, [sv])
        zdv = plsc.load_gather(z_v, [dv])
        pair_v[pl.ds(off, 16)] = zsv * 100 + zdv
        return _

    lax.fori_loop(0, E_PER_TILE // 16, body, 0, unroll=4)
    pltpu.sync_copy(d2_v, d2_hbm.at[pl.ds(base, E_PER_TILE)])
    pltpu.sync_copy(pair_v, pair_hbm.at[pl.ds(base, E_PER_TILE)])


def _edge_meta(pos4, z32, src, dst):
    k = pl.kernel(
        _edge_meta_body,
        out_type=[jax.ShapeDtypeStruct((EP,), jnp.float32),
                  jax.ShapeDtypeStruct((EP,), jnp.int32)],
        mesh=_mesh(),
        compiler_params=pltpu.CompilerParams(needs_layout_passes=False, use_tc_tiling_on_sc=False),
        scratch_types=[
            pltpu.VMEM((N_NODES * 4,), jnp.float32),
            pltpu.VMEM((N_NODES,), jnp.int32),
            pltpu.VMEM((E_PER_TILE,), jnp.int32),
            pltpu.VMEM((E_PER_TILE,), jnp.int32),
            pltpu.VMEM((E_PER_TILE,), jnp.float32),
            pltpu.VMEM((E_PER_TILE,), jnp.int32),
        ],
    )
    return k(pos4, z32, src, dst)


_CHUNK = 256
_NSUB = _CHUNK // 128
_BIG = 1024
_ROWS_T = NPAD // 16
_ZROWS = _ROWS_T // 4


def _edge_pass_body(m_hbm, t_hbm, g_hbm, src_hbm, pair_hbm, dst_hbm, out_hbm,
                    acc, src_v, pair_v, dst_v, g_v, m_v, t_v, zbuf,
                    sem_a, sem_b, sem_c):
    cid = lax.axis_index("c")
    sid = lax.axis_index("s")
    wid = sid * 2 + cid
    base = wid * E_PER_TILE

    def zb(i, _):
        zbuf[i >> 2, pl.ds((i & 3) * 16, 16)] = jnp.zeros((16,), jnp.float32)
        return _
    lax.fori_loop(0, _ZROWS * 4, zb, 0, unroll=8)
    row0 = pl.multiple_of(sid * _ROWS_T, _ROWS_T)
    for q in range(4):
        pltpu.sync_copy(zbuf, acc.at[pl.ds(row0 + q * _ZROWS, _ZROWS)])
    plsc.subcore_barrier()

    nbig = E_PER_TILE // _BIG

    def big(ci, _):
        bbase = pl.multiple_of(base + ci * _BIG, _BIG)
        crow = pl.multiple_of(wid * (E_PER_TILE // 128) + ci * 8, 8)
        pltpu.sync_copy(src_hbm.at[pl.ds(crow, 8)], src_v)
        pltpu.sync_copy(pair_hbm.at[pl.ds(crow, 8)], pair_v)
        pltpu.sync_copy(dst_hbm.at[pl.ds(crow, 8)], dst_v)
        for h in range(_BIG // _CHUNK):
            ebase = pl.multiple_of(bbase + h * _CHUNK, _CHUNK)
            dg = pltpu.async_copy(g_hbm.at[pl.ds(ebase, _CHUNK)], g_v, sem_c)
            descs = []
            for j in range(_NSUB):
                jj = h * _NSUB + j
                descs.append(pltpu.async_copy(
                    m_hbm.at[src_v.at[jj]], m_v.at[pl.ds(j * 128, 128)],
                    sem_a))
                descs.append(pltpu.async_copy(
                    t_hbm.at[pair_v.at[jj]], t_v.at[pl.ds(j * 128, 128)],
                    sem_b))
            dg.wait()
            for d in descs:
                d.wait()

            def mul(i, _):
                r = i >> 2
                col = (i & 3) * 16
                mv = m_v[r, pl.ds(col, 16)]
                tv = t_v[r, pl.ds(col, 16)]
                gv = g_v[r, pl.ds(col, 16)]
                m_v[r, pl.ds(col, 16)] = mv * tv * gv
                return _
            lax.fori_loop(0, _CHUNK * 4, mul, 0, unroll=8)

            for j in range(_NSUB):
                pltpu.sync_copy(m_v.at[pl.ds(j * 128, 128)],
                                acc.at[dst_v.at[h * _NSUB + j]], add=True)
        return _

    lax.fori_loop(0, nbig, big, 0)
    plsc.subcore_barrier()

    for q in range(4):
        pltpu.sync_copy(acc.at[pl.ds(row0 + q * _ZROWS, _ZROWS)], zbuf)
        pltpu.sync_copy(zbuf, out_hbm.at[cid, pl.ds(row0 + q * _ZROWS, _ZROWS)])


def _edge_pass(m, t, g, src2, pair2, dst2):
    k = pl.kernel(
        _edge_pass_body,
        out_type=jax.ShapeDtypeStruct((2, NPAD, NF), jnp.float32),
        mesh=_mesh(),
        compiler_params=pltpu.CompilerParams(needs_layout_passes=False, use_tc_tiling_on_sc=False),
        scratch_types=[
            pltpu.VMEM_SHARED((NPAD, NF), jnp.float32),
            pltpu.VMEM((8, 128), jnp.int32),
            pltpu.VMEM((8, 128), jnp.int32),
            pltpu.VMEM((8, 128), jnp.int32),
            pltpu.VMEM((_CHUNK, NF), jnp.float32),
            pltpu.VMEM((_CHUNK, NF), jnp.float32),
            pltpu.VMEM((_CHUNK, NF), jnp.float32),
            pltpu.VMEM((_ZROWS, NF), jnp.float32),
            pltpu.SemaphoreType.DMA,
            pltpu.SemaphoreType.DMA,
            pltpu.SemaphoreType.DMA,
        ],
    )
    return k(m, t, g, src2, pair2, dst2)


def _nuc_body(h_hbm, nuc_hbm, out_hbm, nuc_v, idx_v, rows_v, sem):
    wid = lax.axis_index("s") * 2 + lax.axis_index("c")
    npt = BPAD // 32
    base = pl.multiple_of(wid * npt, npt)
    pltpu.sync_copy(nuc_hbm.at[pl.ds(base, npt)], nuc_v)

    def body(g, _):
        v = nuc_v[pl.ds(g * 16, 16)]
        wrap = jnp.where(v == 0, jnp.full((16,), N_NODES, jnp.int32),
                         jnp.zeros((16,), jnp.int32))
        idx_v[pl.ds(g * 16, 16)] = v - 1 + wrap
        return _
    lax.fori_loop(0, npt // 16, body, 0)
    pltpu.async_copy(h_hbm.at[idx_v], rows_v, sem).wait()
    pltpu.sync_copy(rows_v, out_hbm.at[pl.ds(base, npt)])


def _nuc_gather(h, nucpad):
    npt = BPAD // 32
    k = pl.kernel(
        _nuc_body,
        out_type=jax.ShapeDtypeStruct((BPAD, NF), jnp.float32),
        mesh=_mesh(),
        compiler_params=pltpu.CompilerParams(needs_layout_passes=False, use_tc_tiling_on_sc=False),
        scratch_types=[
            pltpu.VMEM((npt,), jnp.int32),
            pltpu.VMEM((npt,), jnp.int32),
            pltpu.VMEM((npt, NF), jnp.float32),
            pltpu.SemaphoreType.DMA,
        ],
    )
    return k(h, nucpad)


def _prelude_body(e1_ref, zw1t1_ref, zb1_1_ref, zw2t1_ref, zb2_1_ref,
                  e2_ref, zw1t2_ref, zb1_2_ref, zw2t2_ref, zb2_2_ref,
                  emb0_ref, z2_ref, l1wt_ref, l1b_ref,
                  pos27_ref, ang_ref, mor_ref, ssel_ref, rsel_ref,
                  sw1a_ref, sw1d_ref, sw1m_ref, sb1_ref,
                  sw2t_ref, sb2_ref, sw3t_ref, sb3_ref,
                  t1_ref, t2_ref, h0_ref, m1_ref, s_ref):
    def table(e_ref, w1t, b1, w2t, b2):
        e = e_ref[...]
        prod = (e[:, None, :] * e[None, :, :]).reshape(10000, NF)
        t = _sp(jnp.dot(prod, w1t[...], preferred_element_type=jnp.float32)
                + b1[...])
        return _sp(jnp.dot(t, w2t[...], preferred_element_type=jnp.float32)
                   + b2[...])

    t1_ref[...] = table(e1_ref, zw1t1_ref, zb1_1_ref, zw2t1_ref, zb2_1_ref)
    t2_ref[...] = table(e2_ref, zw1t2_ref, zb1_2_ref, zw2t2_ref, zb2_2_ref)

    zc = z2_ref[...]
    iota = lax.broadcasted_iota(jnp.int32, (N_NODES, 100), 1)
    onehot = (iota == zc).astype(jnp.float32)
    h0 = jnp.dot(onehot, emb0_ref[...], preferred_element_type=jnp.float32)
    h0_ref[...] = h0
    m1_ref[...] = jnp.dot(h0, l1wt_ref[...],
                          preferred_element_type=jnp.float32) + l1b_ref[...]

    dp = jnp.dot(pos27_ref[...], ssel_ref[...],
                 preferred_element_type=jnp.float32)
    dist2 = jnp.dot(dp * dp, rsel_ref[...],
                    preferred_element_type=jnp.float32)
    dist = jnp.sqrt(dist2 + 1e-12)
    s1 = (jnp.dot(ang_ref[...], sw1a_ref[...], preferred_element_type=jnp.float32)
          + jnp.dot(dist, sw1d_ref[...], preferred_element_type=jnp.float32)
          + jnp.dot(mor_ref[...], sw1m_ref[...], preferred_element_type=jnp.float32)
          + sb1_ref[...])
    s2 = jnp.dot(s1, sw2t_ref[...], preferred_element_type=jnp.float32) + sb2_ref[...]
    s_ref[...] = jnp.dot(_sp(s2), sw3t_ref[...],
                         preferred_element_type=jnp.float32) + sb3_ref[...]


def _prelude(args):
    return pl.pallas_call(
        _prelude_body,
        out_shape=[
            jax.ShapeDtypeStruct((10000, NF), jnp.float32),
            jax.ShapeDtypeStruct((10000, NF), jnp.float32),
            jax.ShapeDtypeStruct((N_NODES, NF), jnp.float32),
            jax.ShapeDtypeStruct((N_NODES, NF), jnp.float32),
            jax.ShapeDtypeStruct((B, 128), jnp.float32),
        ],
    )(*args)


_GBLK = 2048


def _gfilt_body(d2_ref, w1a_ref, b1a_ref, w2a_ref, b2a_ref,
                w1b_ref, b1b_ref, w2b_ref, b2b_ref, ga_ref, gb_ref):
    gi = pl.program_id(0)
    d = jnp.sqrt(d2_ref[...].reshape(_GBLK, 1) + 1e-12)
    u = lax.broadcasted_iota(jnp.int32, (1, 50), 1).astype(jnp.float32) * 0.1
    diff = d - u
    graw = jnp.exp(-10.0 * diff * diff)
    rows = gi * _GBLK + lax.broadcasted_iota(jnp.int32, (_GBLK, 1), 0)
    valid = rows < N_EDGES

    def filt(w1t, b1, w2t, b2):
        t = _sp(jnp.dot(graw, w1t[...], preferred_element_type=jnp.float32)
                + b1[...])
        g = _sp(jnp.dot(t, w2t[...], preferred_element_type=jnp.float32)
                + b2[...])
        return jnp.where(valid, g, 0.0)

    ga_ref[...] = filt(w1a_ref, b1a_ref, w2a_ref, b2a_ref)
    gb_ref[...] = filt(w1b_ref, b1b_ref, w2b_ref, b2b_ref)


def _gfilt(d2col, wa, ba, wa2, ba2, wb, bb, wb2, bb2):
    nblk = EP // _GBLK
    wspec = pl.BlockSpec((50, NF), lambda i: (0, 0))
    w2spec = pl.BlockSpec((NF, NF), lambda i: (0, 0))
    bspec = pl.BlockSpec((1, NF), lambda i: (0, 0))
    return pl.pallas_call(
        _gfilt_body,
        grid=(nblk,),
        in_specs=[pl.BlockSpec((_GBLK,), lambda i: (i,)),
                  wspec, bspec, w2spec, bspec,
                  wspec, bspec, w2spec, bspec],
        out_specs=[pl.BlockSpec((_GBLK, NF), lambda i: (i, 0)),
                   pl.BlockSpec((_GBLK, NF), lambda i: (i, 0))],
        out_shape=[jax.ShapeDtypeStruct((EP, NF), jnp.float32),
                   jax.ShapeDtypeStruct((EP, NF), jnp.float32)],
    )(d2col, wa, ba, wa2, ba2, wb, bb, wb2, bb2)


def _node_body(vp_ref, h_ref, w1t_ref, b1_ref, w2t_ref, b2_ref,
               nwt_ref, nb_ref, h_out, m_out):
    v = vp_ref[0] + vp_ref[1]
    v = v[0:N_NODES]
    t = _sp(jnp.dot(v, w1t_ref[...], preferred_element_type=jnp.float32)
            + b1_ref[...])
    v2 = jnp.dot(t, w2t_ref[...], preferred_element_type=jnp.float32) + b2_ref[...]
    h = h_ref[...] + v2
    h_out[...] = h
    m_out[...] = jnp.dot(h, nwt_ref[...],
                         preferred_element_type=jnp.float32) + nb_ref[...]


def _node(vp, h, w1t, b1, w2t, b2, nwt, nb):
    return pl.pallas_call(
        _node_body,
        out_shape=[jax.ShapeDtypeStruct((N_NODES, NF), jnp.float32),
                   jax.ShapeDtypeStruct((N_NODES, NF), jnp.float32)],
    )(vp, h, w1t, b1, w2t, b2, nwt, nb)


def _head_body(hx_ref, s_ref, pw1t_ref, pb1_ref, pw2t_ref, pb2_ref,
               aw_ref, bw_ref, hb1_ref, w2t_ref, hb2_ref, w3t_ref, hb3_ref,
               o_ref):
    hx = hx_ref[0:B]
    t = _sp(jnp.dot(hx, pw1t_ref[...], preferred_element_type=jnp.float32)
            + pb1_ref[...])
    hp = jnp.dot(t, pw2t_ref[...], preferred_element_type=jnp.float32) + pb2_ref[...]
    o1 = _sp(jnp.dot(hp, aw_ref[...], preferred_element_type=jnp.float32)
             + jnp.dot(s_ref[...], bw_ref[...], preferred_element_type=jnp.float32)
             + hb1_ref[...])
    o2 = _sp(jnp.dot(o1, w2t_ref[...], preferred_element_type=jnp.float32)
             + hb2_ref[...])
    o_ref[...] = jnp.dot(o2, w3t_ref[...],
                         preferred_element_type=jnp.float32) + hb3_ref[...]


def _head(args):
    return pl.pallas_call(
        _head_body,
        out_shape=jax.ShapeDtypeStruct((B, 1), jnp.float32),
    )(*args)


def kernel(x, pos, angle, keylen, hook, morse, params, edge_index, z, batch,
           nuc_index):
    p = params
    f32 = jnp.float32
    i32 = jnp.int32

    ei = edge_index.astype(i32)
    pad = EP - N_EDGES
    src = jnp.concatenate([ei[0], jnp.zeros((pad,), i32)])
    dst = jnp.concatenate([ei[1], jnp.zeros((pad,), i32)])
    z32 = z.astype(i32)
    pos4 = jnp.concatenate([pos, jnp.zeros((N_NODES, 1), f32)],
                           axis=1).reshape(-1)
    nucpad = jnp.concatenate(
        [nuc_index.astype(i32), jnp.ones((BPAD - B,), i32)])

    iu, ju = np.triu_indices(9, k=1)
    S = np.zeros((27, 108), np.float32)
    R = np.zeros((108, 36), np.float32)
    for k in range(36):
        for c in range(3):
            S[3 * iu[k] + c, 3 * k + c] = 1.0
            S[3 * ju[k] + c, 3 * k + c] = -1.0
            R[3 * k + c, k] = 1.0
    Ssel = jnp.asarray(S)
    Rsel = jnp.asarray(R)

    i1, i2 = p["inter"]

    def row(b):
        return b.reshape(1, -1)

    prelude_args = (
        i1["emb_z"], i1["z_W1"].T, row(i1["z_b1"]), i1["z_W2"].T, row(i1["z_b2"]),
        i2["emb_z"], i2["z_W1"].T, row(i2["z_b1"]), i2["z_W2"].T, row(i2["z_b2"]),
        p["emb_z"], z32.reshape(N_NODES, 1), i1["lin1_W"].T, row(i1["lin1_b"]),
        pos.reshape(B, 27), angle, morse, Ssel, Rsel,
        p["solv_W1"].T[0:8], p["solv_W1"].T[8:44], p["solv_W1"].T[44:52],
        row(p["solv_b1"]),
        p["solv_W2"].T, row(p["solv_b2"]), p["solv_W3"].T, row(p["solv_b3"]),
    )
    t1, t2, h0, m1, s = _prelude(prelude_args)

    d2, pair = _edge_meta(pos4, z32, src, dst)

    ga, gb = _gfilt(d2,
                    i1["g_W1"].T, row(i1["g_b1"]), i1["g_W2"].T, row(i1["g_b2"]),
                    i2["g_W1"].T, row(i2["g_b1"]), i2["g_W2"].T, row(i2["g_b2"]))

    src2 = src.reshape(EP // 128, 128)
    pair2 = pair.reshape(EP // 128, 128)
    dst2 = dst.reshape(EP // 128, 128)

    vp1 = _edge_pass(m1, t1, ga, src2, pair2, dst2)
    h1, m2 = _node(vp1, h0, i1["mlp_W1"].T, row(i1["mlp_b1"]),
                   i1["mlp_W2"].T, row(i1["mlp_b2"]),
                   i2["lin1_W"].T, row(i2["lin1_b"]))

    vp2 = _edge_pass(m2, t2, gb, src2, pair2, dst2)
    h2, _unused = _node(vp2, h1, i2["mlp_W1"].T, row(i2["mlp_b1"]),
                        i2["mlp_W2"].T, row(i2["mlp_b2"]),
                        p["post_W1"].T, row(p["post_b1"]))

    hx = _nuc_gather(h2, nucpad)

    head_args = (
        hx, s,
        p["post_W1"].T, row(p["post_b1"]), p["post_W2"].T, row(p["post_b2"]),
        p["p2_W1"].T[0:NF], p["p2_W1"].T[NF:192], row(p["p2_b1"]),
        p["p2_W2"].T, row(p["p2_b2"]), p["p2_W3"].T, row(p["p2_b3"]),
    )
    return _head(head_args)


# speedup vs baseline: 7.8334x; 1.3789x over previous
import functools

import jax
import jax.numpy as jnp
import numpy as np
from jax import lax
from jax.experimental import pallas as pl
from jax.experimental.pallas import tpu as pltpu
from jax.experimental.pallas import tpu_sc as plsc

N_NODES = 10008
N_EDGES = 320256
B = 1112
NF = 64

EP = 327680
E_PER_TILE = EP // 32
NPAD = 10240
BPAD = 1536

_LN2 = float(np.log(2.0))


def _sp(x):
    return jnp.maximum(x, 0.0) + jnp.log1p(jnp.exp(-jnp.abs(x))) - _LN2


def _mesh():
    return plsc.VectorSubcoreMesh(core_axis_name="c", subcore_axis_name="s")


def _edge_meta_body(pos_hbm, z_hbm, src_hbm, dst_hbm, d2_hbm, pair_hbm,
                    pos_v, z_v, src_v, dst_v, d2_v, pair_v):
    wid = lax.axis_index("s") * 2 + lax.axis_index("c")
    base = pl.multiple_of(wid * E_PER_TILE, E_PER_TILE)
    pltpu.sync_copy(pos_hbm, pos_v)
    pltpu.sync_copy(z_hbm, z_v)
    pltpu.sync_copy(src_hbm.at[pl.ds(base, E_PER_TILE)], src_v)
    pltpu.sync_copy(dst_hbm.at[pl.ds(base, E_PER_TILE)], dst_v)

    def body(g, _):
        off = g * 16
        sv = src_v[pl.ds(off, 16)]
        dv = dst_v[pl.ds(off, 16)]
        s4 = sv * 4
        d4 = dv * 4
        xs = plsc.load_gather(pos_v, [s4])
        ys = plsc.load_gather(pos_v, [s4 + 1])
        zs = plsc.load_gather(pos_v, [s4 + 2])
        xd = plsc.load_gather(pos_v, [d4])
        yd = plsc.load_gather(pos_v, [d4 + 1])
        zd = plsc.load_gather(pos_v, [d4 + 2])
        dx = xs - xd
        dy = ys - yd
        dz = zs - zd
        d2_v[pl.ds(off, 16)] = dx * dx + dy * dy + dz * dz
        zsv = plsc.load_gather(z_v, [sv])
        zdv = plsc.load_gather(z_v, [dv])
        pair_v[pl.ds(off, 16)] = zsv * 100 + zdv
        return _

    lax.fori_loop(0, E_PER_TILE // 16, body, 0, unroll=4)
    pltpu.sync_copy(d2_v, d2_hbm.at[pl.ds(base, E_PER_TILE)])
    pltpu.sync_copy(pair_v, pair_hbm.at[pl.ds(base, E_PER_TILE)])


def _edge_meta(pos4, z32, src, dst):
    k = pl.kernel(
        _edge_meta_body,
        out_type=[jax.ShapeDtypeStruct((EP,), jnp.float32),
                  jax.ShapeDtypeStruct((EP,), jnp.int32)],
        mesh=_mesh(),
        compiler_params=pltpu.CompilerParams(needs_layout_passes=False, use_tc_tiling_on_sc=False),
        scratch_types=[
            pltpu.VMEM((N_NODES * 4,), jnp.float32),
            pltpu.VMEM((N_NODES,), jnp.int32),
            pltpu.VMEM((E_PER_TILE,), jnp.int32),
            pltpu.VMEM((E_PER_TILE,), jnp.int32),
            pltpu.VMEM((E_PER_TILE,), jnp.float32),
            pltpu.VMEM((E_PER_TILE,), jnp.int32),
        ],
    )
    return k(pos4, z32, src, dst)


_CHUNK = 128
_BIG = 1024
_NBB = E_PER_TILE // _BIG
_ROWS_T = NPAD // 16
_ZROWS = _ROWS_T // 4


def _edge_pass_body(goff, m_hbm, t_hbm, g_hbm, src_hbm, pair_hbm, dst_hbm,
                    out_hbm, acc, src_v, pair_v, dst_v, g_v, m_v, t_v, zbuf,
                    sem_a, sem_b, sem_c):
    cid = lax.axis_index("c")
    sid = lax.axis_index("s")
    wid = sid * 2 + cid
    base = wid * E_PER_TILE

    def zb(i, _):
        zbuf[i >> 2, pl.ds((i & 3) * 16, 16)] = jnp.zeros((16,), jnp.float32)
        return _
    lax.fori_loop(0, _ZROWS * 4, zb, 0, unroll=8)
    row0 = pl.multiple_of(sid * _ROWS_T, _ROWS_T)
    for q in range(4):
        pltpu.sync_copy(zbuf, acc.at[pl.ds(row0 + q * _ZROWS, _ZROWS)])
    plsc.subcore_barrier()

    def load_idx(bb):
        crow = pl.multiple_of(wid * (E_PER_TILE // 128) + bb * 8, 8)
        pltpu.sync_copy(src_hbm.at[pl.ds(crow, 8)], src_v)
        pltpu.sync_copy(pair_hbm.at[pl.ds(crow, 8)], pair_v)
        pltpu.sync_copy(dst_hbm.at[pl.ds(crow, 8)], dst_v)

    def fire(bb, k, buf):
        ebase = pl.multiple_of(base + bb * _BIG + k * _CHUNK, _CHUNK)
        pltpu.async_copy(g_hbm.at[pl.ds(ebase, _CHUNK), pl.ds(goff, NF)],
                         g_v.at[buf], sem_c)
        pltpu.async_copy(m_hbm.at[src_v.at[k]], m_v.at[buf], sem_a)
        pltpu.async_copy(t_hbm.at[pair_v.at[k]], t_v.at[buf], sem_b)

    def drain(buf):
        pltpu.make_async_copy(g_hbm.at[pl.ds(0, _CHUNK), pl.ds(goff, NF)],
                              g_v.at[buf], sem_c).wait()
        pltpu.make_async_copy(m_hbm.at[pl.ds(0, _CHUNK)], m_v.at[buf],
                              sem_a).wait()
        pltpu.make_async_copy(t_hbm.at[pl.ds(0, _CHUNK)], t_v.at[buf],
                              sem_b).wait()

    load_idx(0)
    fire(0, 0, 0)

    def big(bb, _):
        for k in range(_BIG // _CHUNK):
            buf = k & 1
            if k < _BIG // _CHUNK - 1:
                fire(bb, k + 1, 1 - buf)
            drain(buf)

            def mul(i, _):
                r = i >> 2
                col = (i & 3) * 16
                mv = m_v[buf, r, pl.ds(col, 16)]
                tv = t_v[buf, r, pl.ds(col, 16)]
                gv = g_v[buf, r, pl.ds(col, 16)]
                m_v[buf, r, pl.ds(col, 16)] = mv * tv * gv
                return _
            lax.fori_loop(0, _CHUNK * 4, mul, 0, unroll=8)

            pltpu.sync_copy(m_v.at[buf], acc.at[dst_v.at[k]], add=True)
            if k == _BIG // _CHUNK - 1:
                @pl.when(bb + 1 < _NBB)
                def _prime_next():
                    load_idx(bb + 1)
                    fire(bb + 1, 0, 0)
        return _

    lax.fori_loop(0, _NBB, big, 0)
    plsc.subcore_barrier()

    for q in range(4):
        pltpu.sync_copy(acc.at[pl.ds(row0 + q * _ZROWS, _ZROWS)], zbuf)
        pltpu.sync_copy(zbuf, out_hbm.at[cid, pl.ds(row0 + q * _ZROWS, _ZROWS)])


def _edge_pass(m, t, g, src2, pair2, dst2, goff):
    k = pl.kernel(
        functools.partial(_edge_pass_body, goff),
        out_type=jax.ShapeDtypeStruct((2, NPAD, NF), jnp.float32),
        mesh=_mesh(),
        compiler_params=pltpu.CompilerParams(needs_layout_passes=False, use_tc_tiling_on_sc=False),
        scratch_types=[
            pltpu.VMEM_SHARED((NPAD, NF), jnp.float32),
            pltpu.VMEM((8, 128), jnp.int32),
            pltpu.VMEM((8, 128), jnp.int32),
            pltpu.VMEM((8, 128), jnp.int32),
            pltpu.VMEM((2, _CHUNK, NF), jnp.float32),
            pltpu.VMEM((2, _CHUNK, NF), jnp.float32),
            pltpu.VMEM((2, _CHUNK, NF), jnp.float32),
            pltpu.VMEM((_ZROWS, NF), jnp.float32),
            pltpu.SemaphoreType.DMA,
            pltpu.SemaphoreType.DMA,
            pltpu.SemaphoreType.DMA,
        ],
    )
    return k(m, t, g, src2, pair2, dst2)


def _nuc_body(h_hbm, nuc_hbm, out_hbm, nuc_v, idx_v, rows_v, sem):
    wid = lax.axis_index("s") * 2 + lax.axis_index("c")
    npt = BPAD // 32
    base = pl.multiple_of(wid * npt, npt)
    pltpu.sync_copy(nuc_hbm.at[pl.ds(base, npt)], nuc_v)

    def body(g, _):
        v = nuc_v[pl.ds(g * 16, 16)]
        wrap = jnp.where(v == 0, jnp.full((16,), N_NODES, jnp.int32),
                         jnp.zeros((16,), jnp.int32))
        idx_v[pl.ds(g * 16, 16)] = v - 1 + wrap
        return _
    lax.fori_loop(0, npt // 16, body, 0)
    pltpu.async_copy(h_hbm.at[idx_v], rows_v, sem).wait()
    pltpu.sync_copy(rows_v, out_hbm.at[pl.ds(base, npt)])


def _nuc_gather(h, nucpad):
    npt = BPAD // 32
    k = pl.kernel(
        _nuc_body,
        out_type=jax.ShapeDtypeStruct((BPAD, NF), jnp.float32),
        mesh=_mesh(),
        compiler_params=pltpu.CompilerParams(needs_layout_passes=False, use_tc_tiling_on_sc=False),
        scratch_types=[
            pltpu.VMEM((npt,), jnp.int32),
            pltpu.VMEM((npt,), jnp.int32),
            pltpu.VMEM((npt, NF), jnp.float32),
            pltpu.SemaphoreType.DMA,
        ],
    )
    return k(h, nucpad)


def _prelude_body(e1_ref, zw1t1_ref, zb1_1_ref, zw2t1_ref, zb2_1_ref,
                  e2_ref, zw1t2_ref, zb1_2_ref, zw2t2_ref, zb2_2_ref,
                  emb0_ref, z2_ref, l1wt_ref, l1b_ref,
                  pos27_ref, ang_ref, mor_ref, ssel_ref, rsel_ref,
                  sw1a_ref, sw1d_ref, sw1m_ref, sb1_ref,
                  sw2t_ref, sb2_ref, sw3t_ref, sb3_ref,
                  t1_ref, t2_ref, h0_ref, m1_ref, s_ref):
    def table(e_ref, w1t, b1, w2t, b2):
        e = e_ref[...]
        prod = (e[:, None, :] * e[None, :, :]).reshape(10000, NF)
        t = _sp(jnp.dot(prod, w1t[...], preferred_element_type=jnp.float32)
                + b1[...])
        return _sp(jnp.dot(t, w2t[...], preferred_element_type=jnp.float32)
                   + b2[...])

    t1_ref[...] = table(e1_ref, zw1t1_ref, zb1_1_ref, zw2t1_ref, zb2_1_ref)
    t2_ref[...] = table(e2_ref, zw1t2_ref, zb1_2_ref, zw2t2_ref, zb2_2_ref)

    zc = z2_ref[...]
    iota = lax.broadcasted_iota(jnp.int32, (N_NODES, 100), 1)
    onehot = (iota == zc).astype(jnp.float32)
    h0 = jnp.dot(onehot, emb0_ref[...], preferred_element_type=jnp.float32)
    h0_ref[...] = h0
    m1_ref[...] = jnp.dot(h0, l1wt_ref[...],
                          preferred_element_type=jnp.float32) + l1b_ref[...]

    dp = jnp.dot(pos27_ref[...], ssel_ref[...],
                 preferred_element_type=jnp.float32)
    dist2 = jnp.dot(dp * dp, rsel_ref[...],
                    preferred_element_type=jnp.float32)
    dist = jnp.sqrt(dist2 + 1e-12)
    s1 = (jnp.dot(ang_ref[...], sw1a_ref[...], preferred_element_type=jnp.float32)
          + jnp.dot(dist, sw1d_ref[...], preferred_element_type=jnp.float32)
          + jnp.dot(mor_ref[...], sw1m_ref[...], preferred_element_type=jnp.float32)
          + sb1_ref[...])
    s2 = jnp.dot(s1, sw2t_ref[...], preferred_element_type=jnp.float32) + sb2_ref[...]
    s_ref[...] = jnp.dot(_sp(s2), sw3t_ref[...],
                         preferred_element_type=jnp.float32) + sb3_ref[...]


def _prelude(args):
    return pl.pallas_call(
        _prelude_body,
        out_shape=[
            jax.ShapeDtypeStruct((10000, NF), jnp.float32),
            jax.ShapeDtypeStruct((10000, NF), jnp.float32),
            jax.ShapeDtypeStruct((N_NODES, NF), jnp.float32),
            jax.ShapeDtypeStruct((N_NODES, NF), jnp.float32),
            jax.ShapeDtypeStruct((B, 128), jnp.float32),
        ],
    )(*args)


_GBLK = 2048


def _gfilt_body(d2_ref, w1a_ref, b1a_ref, w2a_ref, b2a_ref,
                w1b_ref, b1b_ref, w2b_ref, b2b_ref, gout_ref):
    gi = pl.program_id(0)
    d = jnp.sqrt(d2_ref[...].reshape(_GBLK, 1) + 1e-12)
    u = lax.broadcasted_iota(jnp.int32, (1, 50), 1).astype(jnp.float32) * 0.1
    diff = d - u
    graw = jnp.exp(-10.0 * diff * diff)
    rows = gi * _GBLK + lax.broadcasted_iota(jnp.int32, (_GBLK, 1), 0)
    valid = rows < N_EDGES

    def filt(w1t, b1, w2t, b2):
        t = _sp(jnp.dot(graw, w1t[...], preferred_element_type=jnp.float32)
                + b1[...])
        g = _sp(jnp.dot(t, w2t[...], preferred_element_type=jnp.float32)
                + b2[...])
        return jnp.where(valid, g, 0.0)

    gout_ref[...] = jnp.concatenate(
        [filt(w1a_ref, b1a_ref, w2a_ref, b2a_ref),
         filt(w1b_ref, b1b_ref, w2b_ref, b2b_ref)], axis=1)


def _gfilt(d2col, wa, ba, wa2, ba2, wb, bb, wb2, bb2):
    nblk = EP // _GBLK
    wspec = pl.BlockSpec((50, NF), lambda i: (0, 0))
    w2spec = pl.BlockSpec((NF, NF), lambda i: (0, 0))
    bspec = pl.BlockSpec((1, NF), lambda i: (0, 0))
    return pl.pallas_call(
        _gfilt_body,
        grid=(nblk,),
        in_specs=[pl.BlockSpec((_GBLK,), lambda i: (i,)),
                  wspec, bspec, w2spec, bspec,
                  wspec, bspec, w2spec, bspec],
        out_specs=pl.BlockSpec((_GBLK, 128), lambda i: (i, 0)),
        out_shape=jax.ShapeDtypeStruct((EP, 128), jnp.float32),
    )(d2col, wa, ba, wa2, ba2, wb, bb, wb2, bb2)


def _node_body(vp_ref, h_ref, w1t_ref, b1_ref, w2t_ref, b2_ref,
               nwt_ref, nb_ref, h_out, m_out):
    v = vp_ref[0] + vp_ref[1]
    v = v[0:N_NODES]
    t = _sp(jnp.dot(v, w1t_ref[...], preferred_element_type=jnp.float32)
            + b1_ref[...])
    v2 = jnp.dot(t, w2t_ref[...], preferred_element_type=jnp.float32) + b2_ref[...]
    h = h_ref[...] + v2
    h_out[...] = h
    m_out[...] = jnp.dot(h, nwt_ref[...],
                         preferred_element_type=jnp.float32) + nb_ref[...]


def _node(vp, h, w1t, b1, w2t, b2, nwt, nb):
    return pl.pallas_call(
        _node_body,
        out_shape=[jax.ShapeDtypeStruct((N_NODES, NF), jnp.float32),
                   jax.ShapeDtypeStruct((N_NODES, NF), jnp.float32)],
    )(vp, h, w1t, b1, w2t, b2, nwt, nb)


def _head_body(hx_ref, s_ref, pw1t_ref, pb1_ref, pw2t_ref, pb2_ref,
               aw_ref, bw_ref, hb1_ref, w2t_ref, hb2_ref, w3t_ref, hb3_ref,
               o_ref):
    hx = hx_ref[0:B]
    t = _sp(jnp.dot(hx, pw1t_ref[...], preferred_element_type=jnp.float32)
            + pb1_ref[...])
    hp = jnp.dot(t, pw2t_ref[...], preferred_element_type=jnp.float32) + pb2_ref[...]
    o1 = _sp(jnp.dot(hp, aw_ref[...], preferred_element_type=jnp.float32)
             + jnp.dot(s_ref[...], bw_ref[...], preferred_element_type=jnp.float32)
             + hb1_ref[...])
    o2 = _sp(jnp.dot(o1, w2t_ref[...], preferred_element_type=jnp.float32)
             + hb2_ref[...])
    o_ref[...] = jnp.dot(o2, w3t_ref[...],
                         preferred_element_type=jnp.float32) + hb3_ref[...]


def _head(args):
    return pl.pallas_call(
        _head_body,
        out_shape=jax.ShapeDtypeStruct((B, 1), jnp.float32),
    )(*args)


def kernel(x, pos, angle, keylen, hook, morse, params, edge_index, z, batch,
           nuc_index):
    p = params
    f32 = jnp.float32
    i32 = jnp.int32

    ei = edge_index.astype(i32)
    pad = EP - N_EDGES
    src = jnp.concatenate([ei[0], jnp.zeros((pad,), i32)])
    dst = jnp.concatenate([ei[1], jnp.zeros((pad,), i32)])
    z32 = z.astype(i32)
    pos4 = jnp.concatenate([pos, jnp.zeros((N_NODES, 1), f32)],
                           axis=1).reshape(-1)
    nucpad = jnp.concatenate(
        [nuc_index.astype(i32), jnp.ones((BPAD - B,), i32)])

    iu, ju = np.triu_indices(9, k=1)
    S = np.zeros((27, 108), np.float32)
    R = np.zeros((108, 36), np.float32)
    for k in range(36):
        for c in range(3):
            S[3 * iu[k] + c, 3 * k + c] = 1.0
            S[3 * ju[k] + c, 3 * k + c] = -1.0
            R[3 * k + c, k] = 1.0
    Ssel = jnp.asarray(S)
    Rsel = jnp.asarray(R)

    i1, i2 = p["inter"]

    def row(b):
        return b.reshape(1, -1)

    prelude_args = (
        i1["emb_z"], i1["z_W1"].T, row(i1["z_b1"]), i1["z_W2"].T, row(i1["z_b2"]),
        i2["emb_z"], i2["z_W1"].T, row(i2["z_b1"]), i2["z_W2"].T, row(i2["z_b2"]),
        p["emb_z"], z32.reshape(N_NODES, 1), i1["lin1_W"].T, row(i1["lin1_b"]),
        pos.reshape(B, 27), angle, morse, Ssel, Rsel,
        p["solv_W1"].T[0:8], p["solv_W1"].T[8:44], p["solv_W1"].T[44:52],
        row(p["solv_b1"]),
        p["solv_W2"].T, row(p["solv_b2"]), p["solv_W3"].T, row(p["solv_b3"]),
    )
    t1, t2, h0, m1, s = _prelude(prelude_args)

    d2, pair = _edge_meta(pos4, z32, src, dst)

    gboth = _gfilt(d2,
                    i1["g_W1"].T, row(i1["g_b1"]), i1["g_W2"].T, row(i1["g_b2"]),
                    i2["g_W1"].T, row(i2["g_b1"]), i2["g_W2"].T, row(i2["g_b2"]))

    src2 = src.reshape(EP // 128, 128)
    pair2 = pair.reshape(EP // 128, 128)
    dst2 = dst.reshape(EP // 128, 128)

    vp1 = _edge_pass(m1, t1, gboth, src2, pair2, dst2, 0)
    h1, m2 = _node(vp1, h0, i1["mlp_W1"].T, row(i1["mlp_b1"]),
                   i1["mlp_W2"].T, row(i1["mlp_b2"]),
                   i2["lin1_W"].T, row(i2["lin1_b"]))

    vp2 = _edge_pass(m2, t2, gboth, src2, pair2, dst2, NF)
    h2, _unused = _node(vp2, h1, i2["mlp_W1"].T, row(i2["mlp_b1"]),
                        i2["mlp_W2"].T, row(i2["mlp_b2"]),
                        p["post_W1"].T, row(p["post_b1"]))

    hx = _nuc_gather(h2, nucpad)

    head_args = (
        hx, s,
        p["post_W1"].T, row(p["post_b1"]), p["post_W2"].T, row(p["post_b2"]),
        p["p2_W1"].T[0:NF], p["p2_W1"].T[NF:192], row(p["p2_b1"]),
        p["p2_W2"].T, row(p["p2_b2"]), p["p2_W3"].T, row(p["p2_b3"]),
    )
    return _head(head_args)
